# Initial kernel scaffold; baseline (speedup 1.0000x reference)
import jax
import jax.numpy as jnp
from jax.experimental import pallas as pl


def kernel(x, edge_index, W, att_src, att_dst, bias, lin_W, lin_b):
    raise NotImplementedError("write your pallas kernel here")


# speedup vs baseline: 1.0071x; 1.0071x over previous
import jax
import jax.numpy as jnp
from jax.experimental import pallas as pl

_N = 10000
_IN = 128
_C = 512
_H = 8
_NBLK = 1000


def _proj_kernel(x_ref, w_ref, asrc_ref, adst_ref, xw_ref, as_ref, ad_ref):
    xw = jnp.dot(x_ref[...], w_ref[...], preferred_element_type=jnp.float32)
    xw_ref[...] = xw
    as_ref[...] = jnp.dot(xw, asrc_ref[...], preferred_element_type=jnp.float32)
    ad_ref[...] = jnp.dot(xw, adst_ref[...], preferred_element_type=jnp.float32)


def _lin_kernel(agg_ref, bias_ref, linw_ref, linb_ref, out_ref):
    y = agg_ref[...] + bias_ref[...]
    out_ref[...] = jax.lax.dot_general(
        y, linw_ref[...],
        dimension_numbers=(((1,), (1,)), ((), ())),
        preferred_element_type=jnp.float32,
    ) + linb_ref[...]


def kernel(x, edge_index, W, att_src, att_dst, bias, lin_W, lin_b):
    N, H, C = _N, _H, _C
    nblocks = N // _NBLK
    HC = H * C

    head_mask = (jnp.arange(HC)[:, None] // C) == jnp.arange(H)[None, :]
    asrc_bd = jnp.where(head_mask, att_src.reshape(HC, 1), 0.0)
    adst_bd = jnp.where(head_mask, att_dst.reshape(HC, 1), 0.0)

    xw, a_src, a_dst = pl.pallas_call(
        _proj_kernel,
        grid=(nblocks,),
        in_specs=[
            pl.BlockSpec((_NBLK, _IN), lambda i: (i, 0)),
            pl.BlockSpec((_IN, HC), lambda i: (0, 0)),
            pl.BlockSpec((HC, H), lambda i: (0, 0)),
            pl.BlockSpec((HC, H), lambda i: (0, 0)),
        ],
        out_specs=[
            pl.BlockSpec((_NBLK, HC), lambda i: (i, 0)),
            pl.BlockSpec((_NBLK, H), lambda i: (i, 0)),
            pl.BlockSpec((_NBLK, H), lambda i: (i, 0)),
        ],
        out_shape=[
            jax.ShapeDtypeStruct((N, HC), jnp.float32),
            jax.ShapeDtypeStruct((N, H), jnp.float32),
            jax.ShapeDtypeStruct((N, H), jnp.float32),
        ],
    )(x, W, asrc_bd, adst_bd)

    ei = edge_index.astype(jnp.int32)
    loop = jnp.arange(N, dtype=jnp.int32)
    src = jnp.concatenate([ei[0], loop])
    dst = jnp.concatenate([ei[1], loop])
    alpha = a_src[src] + a_dst[dst]
    alpha = jax.nn.leaky_relu(alpha, 0.2)
    amax = jax.ops.segment_max(alpha, dst, num_segments=N)
    alpha = jnp.exp(alpha - amax[dst])
    denom = jax.ops.segment_sum(alpha, dst, num_segments=N)
    alpha = alpha / (denom[dst] + 1e-16)

    xw3 = xw.reshape(N, H, C)
    msg = xw3[src] * alpha[:, :, None]
    agg = jax.ops.segment_sum(msg, dst, num_segments=N).reshape(N, HC)

    out = pl.pallas_call(
        _lin_kernel,
        grid=(nblocks,),
        in_specs=[
            pl.BlockSpec((_NBLK, HC), lambda i: (i, 0)),
            pl.BlockSpec((1, HC), lambda i: (0, 0)),
            pl.BlockSpec((C, HC), lambda i: (0, 0)),
            pl.BlockSpec((1, C), lambda i: (0, 0)),
        ],
        out_specs=pl.BlockSpec((_NBLK, C), lambda i: (i, 0)),
        out_shape=jax.ShapeDtypeStruct((N, C), jnp.float32),
    )(agg, bias.reshape(1, HC), lin_W, lin_b.reshape(1, C))
    return out
